# baseline (speedup 1.0000x reference)
import functools

import jax
import jax.numpy as jnp
from jax import lax
from jax.experimental import pallas as pl
from jax.experimental.pallas import tpu as pltpu
from jax.experimental.pallas import tpu_sc as plsc

B, L, D = 8, 512, 256
F, K, M = 256, 3, 4096
NMB = 4
MBLK = M // NMB


def _ln_relu(h, g, bb):
    mu = jnp.mean(h, axis=-1, keepdims=True)
    var = jnp.mean((h - mu) ** 2, axis=-1, keepdims=True)
    hn = (h - mu) * lax.rsqrt(var + 1e-5) * g + bb
    return jnp.maximum(hn, 0.0)


def _conv3(h, w_ref, bias):
    hb = h.astype(jnp.bfloat16)
    y0 = jnp.dot(hb, w_ref[0].astype(jnp.bfloat16),
                 preferred_element_type=jnp.float32)
    y1 = jnp.dot(hb, w_ref[1].astype(jnp.bfloat16),
                 preferred_element_type=jnp.float32)
    y2 = jnp.dot(hb, w_ref[2].astype(jnp.bfloat16),
                 preferred_element_type=jnp.float32)
    z = jnp.zeros((1, y0.shape[1]), jnp.float32)
    return (jnp.concatenate([z, y0[:-1]], axis=0)
            + y1
            + jnp.concatenate([y2[1:], z], axis=0)
            + bias)


def _tc_body(x_ref, w1_ref, b1_ref, g1_ref, bb1_ref, w2_ref, b2_ref,
             g2_ref, bb2_ref, lw_ref, lb_ref, tgt_ref, tot_ref,
             dur_ref, out_ref, csum_ref):
    b = pl.program_id(0)
    mb = pl.program_id(1)
    xb = x_ref[0]

    @pl.when(mb == 0)
    def _predictor():
        h = _conv3(xb, w1_ref, b1_ref[...])
        h = _ln_relu(h, g1_ref[...], bb1_ref[...])
        h = _conv3(h, w2_ref, b2_ref[...])
        h = _ln_relu(h, g2_ref[...], bb2_ref[...])
        dur = jnp.maximum(
            jnp.dot(h, lw_ref[...], preferred_element_type=jnp.float32)
            + lb_ref[...], 0.0)
        dur_ref[0] = dur
        tgt = tgt_ref[0].astype(jnp.float32)
        ik = lax.broadcasted_iota(jnp.int32, (L, L), 0)
        ij = lax.broadcasted_iota(jnp.int32, (L, L), 1)
        lower_tri = (ik <= ij).astype(jnp.float32)
        csum_ref[...] = jnp.dot(tgt, lower_tri,
                                preferred_element_type=jnp.float32)

    tot = tot_ref[b]

    @pl.when(mb * MBLK < tot)
    def _expand():
        csum = csum_ref[...]
        t_col = (lax.broadcasted_iota(jnp.int32, (MBLK, 1), 0)
                 + mb * MBLK).astype(jnp.float32)
        v = (t_col < csum).astype(jnp.bfloat16)
        z = jnp.zeros((MBLK, 1), jnp.bfloat16)
        onehot = v - jnp.concatenate([z, v[:, :-1]], axis=1)
        out_ref[0] = jnp.dot(onehot, xb.astype(jnp.bfloat16),
                             preferred_element_type=jnp.float32)

    @pl.when(mb * MBLK >= tot)
    def _zero():
        out_ref[0] = jnp.zeros((MBLK, D), jnp.float32)


def kernel(x, conv1_w, conv1_b, ln1_g, ln1_b, conv2_w, conv2_b, ln2_g, ln2_b,
           lin_w, lin_b, target, mel_max_length):
    del mel_max_length
    row = lambda v: v.reshape(1, -1)
    dur3, out = pl.pallas_call(
        _tc_body,
        grid=(B, NMB),
        in_specs=[
            pl.BlockSpec((1, L, D), lambda b, mb: (b, 0, 0)),
            pl.BlockSpec((K, D, F), lambda b, mb: (0, 0, 0)),
            pl.BlockSpec((1, F), lambda b, mb: (0, 0)),
            pl.BlockSpec((1, F), lambda b, mb: (0, 0)),
            pl.BlockSpec((1, F), lambda b, mb: (0, 0)),
            pl.BlockSpec((K, F, F), lambda b, mb: (0, 0, 0)),
            pl.BlockSpec((1, F), lambda b, mb: (0, 0)),
            pl.BlockSpec((1, F), lambda b, mb: (0, 0)),
            pl.BlockSpec((1, F), lambda b, mb: (0, 0)),
            pl.BlockSpec((F, 1), lambda b, mb: (0, 0)),
            pl.BlockSpec((1, 1), lambda b, mb: (0, 0)),
            pl.BlockSpec((1, 1, L), lambda b, mb: (b, 0, 0)),
            pl.BlockSpec(memory_space=pltpu.SMEM),
        ],
        out_specs=[
            pl.BlockSpec((1, L, 1), lambda b, mb: (b, 0, 0)),
            pl.BlockSpec((1, MBLK, D), lambda b, mb: (b, mb, 0)),
        ],
        out_shape=[
            jax.ShapeDtypeStruct((B, L, 1), jnp.float32),
            jax.ShapeDtypeStruct((B, M, D), jnp.float32),
        ],
        scratch_shapes=[pltpu.VMEM((1, L), jnp.float32)],
    )(x, conv1_w, row(conv1_b), row(ln1_g), row(ln1_b), conv2_w,
      row(conv2_b), row(ln2_g), row(ln2_b), lin_w, lin_b.reshape(1, 1),
      target.reshape(B, 1, L),
      jnp.sum(target, axis=1, dtype=jnp.int32))
    return (out, dur3.reshape(B, L))


# speedup vs baseline: 1.1533x; 1.1533x over previous
import functools

import jax
import jax.numpy as jnp
from jax import lax
from jax.experimental import pallas as pl
from jax.experimental.pallas import tpu as pltpu
from jax.experimental.pallas import tpu_sc as plsc

B, L, D = 8, 512, 256
F, K, M = 256, 3, 4096
NMB = 2
MBLK = M // NMB


def _ln_relu(h, g, bb):
    mu = jnp.mean(h, axis=-1, keepdims=True)
    var = jnp.mean((h - mu) ** 2, axis=-1, keepdims=True)
    hn = (h - mu) * lax.rsqrt(var + 1e-5) * g + bb
    return jnp.maximum(hn, 0.0)


def _conv3(h, w_ref, bias):
    hb = h.astype(jnp.bfloat16)
    y0 = jnp.dot(hb, w_ref[0].astype(jnp.bfloat16),
                 preferred_element_type=jnp.float32)
    y1 = jnp.dot(hb, w_ref[1].astype(jnp.bfloat16),
                 preferred_element_type=jnp.float32)
    y2 = jnp.dot(hb, w_ref[2].astype(jnp.bfloat16),
                 preferred_element_type=jnp.float32)
    z = jnp.zeros((1, y0.shape[1]), jnp.float32)
    return (jnp.concatenate([z, y0[:-1]], axis=0)
            + y1
            + jnp.concatenate([y2[1:], z], axis=0)
            + bias)


def _tc_body(x_ref, w1_ref, b1_ref, g1_ref, bb1_ref, w2_ref, b2_ref,
             g2_ref, bb2_ref, lw_ref, lb_ref, tgt_ref, tot_ref,
             dur_ref, out_ref, csum_ref):
    b = pl.program_id(0)
    mb = pl.program_id(1)
    xb = x_ref[0]

    @pl.when(mb == 0)
    def _predictor():
        h = _conv3(xb, w1_ref, b1_ref[...])
        h = _ln_relu(h, g1_ref[...], bb1_ref[...])
        h = _conv3(h, w2_ref, b2_ref[...])
        h = _ln_relu(h, g2_ref[...], bb2_ref[...])
        dur = jnp.maximum(
            jnp.dot(h, lw_ref[...], preferred_element_type=jnp.float32)
            + lb_ref[...], 0.0)
        dur_ref[0] = dur
        tgt = tgt_ref[0].astype(jnp.float32)
        ik = lax.broadcasted_iota(jnp.int32, (L, L), 0)
        ij = lax.broadcasted_iota(jnp.int32, (L, L), 1)
        lower_tri = (ik <= ij).astype(jnp.float32)
        csum_ref[...] = jnp.dot(tgt, lower_tri,
                                preferred_element_type=jnp.float32)

    tot = tot_ref[b]

    @pl.when(mb * MBLK < tot)
    def _expand():
        csum = csum_ref[...]
        t_col = (lax.broadcasted_iota(jnp.int32, (MBLK, 1), 0)
                 + mb * MBLK).astype(jnp.float32)
        v = (t_col < csum).astype(jnp.bfloat16)
        z = jnp.zeros((MBLK, 1), jnp.bfloat16)
        onehot = v - jnp.concatenate([z, v[:, :-1]], axis=1)
        out_ref[0] = jnp.dot(onehot, xb.astype(jnp.bfloat16),
                             preferred_element_type=jnp.float32)

    @pl.when(mb * MBLK >= tot)
    def _zero():
        out_ref[0] = jnp.zeros((MBLK, D), jnp.float32)


def kernel(x, conv1_w, conv1_b, ln1_g, ln1_b, conv2_w, conv2_b, ln2_g, ln2_b,
           lin_w, lin_b, target, mel_max_length):
    del mel_max_length
    row = lambda v: v.reshape(1, -1)
    dur3, out = pl.pallas_call(
        _tc_body,
        grid=(B, NMB),
        in_specs=[
            pl.BlockSpec((1, L, D), lambda b, mb: (b, 0, 0)),
            pl.BlockSpec((K, D, F), lambda b, mb: (0, 0, 0)),
            pl.BlockSpec((1, F), lambda b, mb: (0, 0)),
            pl.BlockSpec((1, F), lambda b, mb: (0, 0)),
            pl.BlockSpec((1, F), lambda b, mb: (0, 0)),
            pl.BlockSpec((K, F, F), lambda b, mb: (0, 0, 0)),
            pl.BlockSpec((1, F), lambda b, mb: (0, 0)),
            pl.BlockSpec((1, F), lambda b, mb: (0, 0)),
            pl.BlockSpec((1, F), lambda b, mb: (0, 0)),
            pl.BlockSpec((F, 1), lambda b, mb: (0, 0)),
            pl.BlockSpec((1, 1), lambda b, mb: (0, 0)),
            pl.BlockSpec((1, 1, L), lambda b, mb: (b, 0, 0)),
            pl.BlockSpec(memory_space=pltpu.SMEM),
        ],
        out_specs=[
            pl.BlockSpec((1, L, 1), lambda b, mb: (b, 0, 0)),
            pl.BlockSpec((1, MBLK, D), lambda b, mb: (b, mb, 0)),
        ],
        out_shape=[
            jax.ShapeDtypeStruct((B, L, 1), jnp.float32),
            jax.ShapeDtypeStruct((B, M, D), jnp.float32),
        ],
        scratch_shapes=[pltpu.VMEM((1, L), jnp.float32)],
    )(x, conv1_w, row(conv1_b), row(ln1_g), row(ln1_b), conv2_w,
      row(conv2_b), row(ln2_g), row(ln2_b), lin_w, lin_b.reshape(1, 1),
      target.reshape(B, 1, L),
      jnp.sum(target, axis=1, dtype=jnp.int32))
    return (out, dur3.reshape(B, L))


# speedup vs baseline: 1.4294x; 1.2395x over previous
import functools

import jax
import jax.numpy as jnp
from jax import lax
from jax.experimental import pallas as pl
from jax.experimental.pallas import tpu as pltpu
from jax.experimental.pallas import tpu_sc as plsc

B, L, D = 8, 512, 256
F, K, M = 256, 3, 4096
NMB = 1
MBLK = M // NMB


def _ln_relu(h, g, bb):
    mu = jnp.mean(h, axis=-1, keepdims=True)
    var = jnp.mean((h - mu) ** 2, axis=-1, keepdims=True)
    hn = (h - mu) * lax.rsqrt(var + 1e-5) * g + bb
    return jnp.maximum(hn, 0.0)


def _conv3(h, w_ref, bias):
    hb = h.astype(jnp.bfloat16)
    y0 = jnp.dot(hb, w_ref[0].astype(jnp.bfloat16),
                 preferred_element_type=jnp.float32)
    y1 = jnp.dot(hb, w_ref[1].astype(jnp.bfloat16),
                 preferred_element_type=jnp.float32)
    y2 = jnp.dot(hb, w_ref[2].astype(jnp.bfloat16),
                 preferred_element_type=jnp.float32)
    z = jnp.zeros((1, y0.shape[1]), jnp.float32)
    return (jnp.concatenate([z, y0[:-1]], axis=0)
            + y1
            + jnp.concatenate([y2[1:], z], axis=0)
            + bias)


def _tc_body(x_ref, w1_ref, b1_ref, g1_ref, bb1_ref, w2_ref, b2_ref,
             g2_ref, bb2_ref, lw_ref, lb_ref, tgt_ref, tot_ref,
             dur_ref, out_ref, csum_ref):
    b = pl.program_id(0)
    mb = pl.program_id(1)
    xb = x_ref[0]

    @pl.when(mb == 0)
    def _predictor():
        h = _conv3(xb, w1_ref, b1_ref[...])
        h = _ln_relu(h, g1_ref[...], bb1_ref[...])
        h = _conv3(h, w2_ref, b2_ref[...])
        h = _ln_relu(h, g2_ref[...], bb2_ref[...])
        dur = jnp.maximum(
            jnp.dot(h, lw_ref[...], preferred_element_type=jnp.float32)
            + lb_ref[...], 0.0)
        dur_ref[0] = dur
        tgt = tgt_ref[0].astype(jnp.float32)
        ik = lax.broadcasted_iota(jnp.int32, (L, L), 0)
        ij = lax.broadcasted_iota(jnp.int32, (L, L), 1)
        lower_tri = (ik <= ij).astype(jnp.float32)
        csum_ref[...] = jnp.dot(tgt, lower_tri,
                                preferred_element_type=jnp.float32)

    tot = tot_ref[b]

    @pl.when(mb * MBLK < tot)
    def _expand():
        csum = csum_ref[...]
        t_col = (lax.broadcasted_iota(jnp.int32, (MBLK, 1), 0)
                 + mb * MBLK).astype(jnp.float32)
        v = (t_col < csum).astype(jnp.bfloat16)
        z = jnp.zeros((MBLK, 1), jnp.bfloat16)
        onehot = v - jnp.concatenate([z, v[:, :-1]], axis=1)
        out_ref[0] = jnp.dot(onehot, xb.astype(jnp.bfloat16),
                             preferred_element_type=jnp.float32)

    @pl.when(mb * MBLK >= tot)
    def _zero():
        out_ref[0] = jnp.zeros((MBLK, D), jnp.float32)


def kernel(x, conv1_w, conv1_b, ln1_g, ln1_b, conv2_w, conv2_b, ln2_g, ln2_b,
           lin_w, lin_b, target, mel_max_length):
    del mel_max_length
    row = lambda v: v.reshape(1, -1)
    dur3, out = pl.pallas_call(
        _tc_body,
        grid=(B, NMB),
        in_specs=[
            pl.BlockSpec((1, L, D), lambda b, mb: (b, 0, 0)),
            pl.BlockSpec((K, D, F), lambda b, mb: (0, 0, 0)),
            pl.BlockSpec((1, F), lambda b, mb: (0, 0)),
            pl.BlockSpec((1, F), lambda b, mb: (0, 0)),
            pl.BlockSpec((1, F), lambda b, mb: (0, 0)),
            pl.BlockSpec((K, F, F), lambda b, mb: (0, 0, 0)),
            pl.BlockSpec((1, F), lambda b, mb: (0, 0)),
            pl.BlockSpec((1, F), lambda b, mb: (0, 0)),
            pl.BlockSpec((1, F), lambda b, mb: (0, 0)),
            pl.BlockSpec((F, 1), lambda b, mb: (0, 0)),
            pl.BlockSpec((1, 1), lambda b, mb: (0, 0)),
            pl.BlockSpec((1, 1, L), lambda b, mb: (b, 0, 0)),
            pl.BlockSpec(memory_space=pltpu.SMEM),
        ],
        out_specs=[
            pl.BlockSpec((1, L, 1), lambda b, mb: (b, 0, 0)),
            pl.BlockSpec((1, MBLK, D), lambda b, mb: (b, mb, 0)),
        ],
        out_shape=[
            jax.ShapeDtypeStruct((B, L, 1), jnp.float32),
            jax.ShapeDtypeStruct((B, M, D), jnp.float32),
        ],
        scratch_shapes=[pltpu.VMEM((1, L), jnp.float32)],
    )(x, conv1_w, row(conv1_b), row(ln1_g), row(ln1_b), conv2_w,
      row(conv2_b), row(ln2_g), row(ln2_b), lin_w, lin_b.reshape(1, 1),
      target.reshape(B, 1, L),
      jnp.sum(target, axis=1, dtype=jnp.int32))
    return (out, dur3.reshape(B, L))


# speedup vs baseline: 1.5231x; 1.0655x over previous
import functools

import jax
import jax.numpy as jnp
from jax import lax
from jax.experimental import pallas as pl
from jax.experimental.pallas import tpu as pltpu
from jax.experimental.pallas import tpu_sc as plsc

B, L, D = 8, 512, 256
F, K, M = 256, 3, 4096
NMB = 1
MBLK = M // NMB
SUB = 1024


def _ln_relu(h, g, bb):
    mu = jnp.mean(h, axis=-1, keepdims=True)
    var = jnp.mean((h - mu) ** 2, axis=-1, keepdims=True)
    hn = (h - mu) * lax.rsqrt(var + 1e-5) * g + bb
    return jnp.maximum(hn, 0.0)


def _conv3(h, w_ref, bias):
    hb = h.astype(jnp.bfloat16)
    y0 = jnp.dot(hb, w_ref[0].astype(jnp.bfloat16),
                 preferred_element_type=jnp.float32)
    y1 = jnp.dot(hb, w_ref[1].astype(jnp.bfloat16),
                 preferred_element_type=jnp.float32)
    y2 = jnp.dot(hb, w_ref[2].astype(jnp.bfloat16),
                 preferred_element_type=jnp.float32)
    z = jnp.zeros((1, y0.shape[1]), jnp.float32)
    return (jnp.concatenate([z, y0[:-1]], axis=0)
            + y1
            + jnp.concatenate([y2[1:], z], axis=0)
            + bias)


def _tc_body(x_ref, w1_ref, b1_ref, g1_ref, bb1_ref, w2_ref, b2_ref,
             g2_ref, bb2_ref, lw_ref, lb_ref, tgt_ref, tot_ref,
             dur_ref, out_ref, csum_ref):
    b = pl.program_id(0)
    mb = pl.program_id(1)
    xb = x_ref[0]

    @pl.when(mb == 0)
    def _predictor():
        h = _conv3(xb, w1_ref, b1_ref[...])
        h = _ln_relu(h, g1_ref[...], bb1_ref[...])
        h = _conv3(h, w2_ref, b2_ref[...])
        h = _ln_relu(h, g2_ref[...], bb2_ref[...])
        dur = jnp.maximum(
            jnp.dot(h, lw_ref[...], preferred_element_type=jnp.float32)
            + lb_ref[...], 0.0)
        dur_ref[0] = dur
        tgt = tgt_ref[0].astype(jnp.float32)
        ik = lax.broadcasted_iota(jnp.int32, (L, L), 0)
        ij = lax.broadcasted_iota(jnp.int32, (L, L), 1)
        lower_tri = (ik <= ij).astype(jnp.float32)
        csum_ref[...] = jnp.dot(tgt, lower_tri,
                                preferred_element_type=jnp.float32)

    tot = tot_ref[b]
    xb16 = xb.astype(jnp.bfloat16)
    csum = csum_ref[...]
    for s in range(MBLK // SUB):
        t0 = mb * MBLK + s * SUB

        @pl.when(t0 < tot)
        def _expand():
            t_col = (lax.broadcasted_iota(jnp.int32, (SUB, 1), 0)
                     + t0).astype(jnp.float32)
            v = (t_col < csum).astype(jnp.bfloat16)
            z = jnp.zeros((SUB, 1), jnp.bfloat16)
            onehot = v - jnp.concatenate([z, v[:, :-1]], axis=1)
            out_ref[0, pl.ds(s * SUB, SUB), :] = jnp.dot(
                onehot, xb16, preferred_element_type=jnp.float32)

        @pl.when(t0 >= tot)
        def _zero():
            out_ref[0, pl.ds(s * SUB, SUB), :] = jnp.zeros(
                (SUB, D), jnp.float32)


def kernel(x, conv1_w, conv1_b, ln1_g, ln1_b, conv2_w, conv2_b, ln2_g, ln2_b,
           lin_w, lin_b, target, mel_max_length):
    del mel_max_length
    row = lambda v: v.reshape(1, -1)
    dur3, out = pl.pallas_call(
        _tc_body,
        grid=(B, NMB),
        in_specs=[
            pl.BlockSpec((1, L, D), lambda b, mb: (b, 0, 0)),
            pl.BlockSpec((K, D, F), lambda b, mb: (0, 0, 0)),
            pl.BlockSpec((1, F), lambda b, mb: (0, 0)),
            pl.BlockSpec((1, F), lambda b, mb: (0, 0)),
            pl.BlockSpec((1, F), lambda b, mb: (0, 0)),
            pl.BlockSpec((K, F, F), lambda b, mb: (0, 0, 0)),
            pl.BlockSpec((1, F), lambda b, mb: (0, 0)),
            pl.BlockSpec((1, F), lambda b, mb: (0, 0)),
            pl.BlockSpec((1, F), lambda b, mb: (0, 0)),
            pl.BlockSpec((F, 1), lambda b, mb: (0, 0)),
            pl.BlockSpec((1, 1), lambda b, mb: (0, 0)),
            pl.BlockSpec((1, 1, L), lambda b, mb: (b, 0, 0)),
            pl.BlockSpec(memory_space=pltpu.SMEM),
        ],
        out_specs=[
            pl.BlockSpec((1, L, 1), lambda b, mb: (b, 0, 0)),
            pl.BlockSpec((1, MBLK, D), lambda b, mb: (b, mb, 0)),
        ],
        out_shape=[
            jax.ShapeDtypeStruct((B, L, 1), jnp.float32),
            jax.ShapeDtypeStruct((B, M, D), jnp.float32),
        ],
        scratch_shapes=[pltpu.VMEM((1, L), jnp.float32)],
    )(x, conv1_w, row(conv1_b), row(ln1_g), row(ln1_b), conv2_w,
      row(conv2_b), row(ln2_g), row(ln2_b), lin_w, lin_b.reshape(1, 1),
      target.reshape(B, 1, L),
      jnp.sum(target, axis=1, dtype=jnp.int32))
    return (out, dur3.reshape(B, L))


# speedup vs baseline: 1.5243x; 1.0008x over previous
import jax
import jax.numpy as jnp
from jax import lax
from jax.experimental import pallas as pl
from jax.experimental.pallas import tpu as pltpu

B, L, D = 8, 512, 256
F, K, M = 256, 3, 4096
NMB = 1
MBLK = M // NMB
SUB = 1024


def _ln_relu(h, g, bb):
    mu = jnp.mean(h, axis=-1, keepdims=True)
    var = jnp.mean((h - mu) ** 2, axis=-1, keepdims=True)
    hn = (h - mu) * lax.rsqrt(var + 1e-5) * g + bb
    return jnp.maximum(hn, 0.0)


def _conv3(h, w_ref, bias):
    hb = h.astype(jnp.bfloat16)
    y0 = jnp.dot(hb, w_ref[0].astype(jnp.bfloat16),
                 preferred_element_type=jnp.float32)
    y1 = jnp.dot(hb, w_ref[1].astype(jnp.bfloat16),
                 preferred_element_type=jnp.float32)
    y2 = jnp.dot(hb, w_ref[2].astype(jnp.bfloat16),
                 preferred_element_type=jnp.float32)
    z = jnp.zeros((1, y0.shape[1]), jnp.float32)
    return (jnp.concatenate([z, y0[:-1]], axis=0)
            + y1
            + jnp.concatenate([y2[1:], z], axis=0)
            + bias)


def _tc_body(x_ref, w1_ref, b1_ref, g1_ref, bb1_ref, w2_ref, b2_ref,
             g2_ref, bb2_ref, lw_ref, lb_ref, tgt_ref, tot_ref,
             dur_ref, out_ref, csum_ref):
    b = pl.program_id(0)
    mb = pl.program_id(1)
    xb = x_ref[0]

    @pl.when(mb == 0)
    def _predictor():
        h = _conv3(xb, w1_ref, b1_ref[...])
        h = _ln_relu(h, g1_ref[...], bb1_ref[...])
        h = _conv3(h, w2_ref, b2_ref[...])
        h = _ln_relu(h, g2_ref[...], bb2_ref[...])
        dur = jnp.maximum(
            jnp.dot(h, lw_ref[...], preferred_element_type=jnp.float32)
            + lb_ref[...], 0.0)
        dur_ref[0] = dur
        tgt = tgt_ref[0].astype(jnp.float32)
        ik = lax.broadcasted_iota(jnp.int32, (L, L), 0)
        ij = lax.broadcasted_iota(jnp.int32, (L, L), 1)
        lower_tri = (ik <= ij).astype(jnp.float32)
        csum_ref[...] = jnp.dot(tgt, lower_tri,
                                preferred_element_type=jnp.float32)

    tot = tot_ref[b]
    xb16 = xb.astype(jnp.bfloat16)
    csum = csum_ref[...]
    for s in range(MBLK // SUB):
        t0 = mb * MBLK + s * SUB

        @pl.when(t0 < tot)
        def _expand():
            t_col = (lax.broadcasted_iota(jnp.int32, (SUB, 1), 0)
                     + t0).astype(jnp.float32)
            v = (t_col < csum).astype(jnp.bfloat16)
            z = jnp.zeros((SUB, 1), jnp.bfloat16)
            onehot = v - jnp.concatenate([z, v[:, :-1]], axis=1)
            out_ref[0, pl.ds(s * SUB, SUB), :] = jnp.dot(
                onehot, xb16, preferred_element_type=jnp.float32)

        @pl.when(t0 >= tot)
        def _zero():
            out_ref[0, pl.ds(s * SUB, SUB), :] = jnp.zeros(
                (SUB, D), jnp.float32)


def kernel(x, conv1_w, conv1_b, ln1_g, ln1_b, conv2_w, conv2_b, ln2_g, ln2_b,
           lin_w, lin_b, target, mel_max_length):
    del mel_max_length
    row = lambda v: v.reshape(1, -1)
    dur3, out = pl.pallas_call(
        _tc_body,
        grid=(B, NMB),
        in_specs=[
            pl.BlockSpec((1, L, D), lambda b, mb: (b, 0, 0)),
            pl.BlockSpec((K, D, F), lambda b, mb: (0, 0, 0)),
            pl.BlockSpec((1, F), lambda b, mb: (0, 0)),
            pl.BlockSpec((1, F), lambda b, mb: (0, 0)),
            pl.BlockSpec((1, F), lambda b, mb: (0, 0)),
            pl.BlockSpec((K, F, F), lambda b, mb: (0, 0, 0)),
            pl.BlockSpec((1, F), lambda b, mb: (0, 0)),
            pl.BlockSpec((1, F), lambda b, mb: (0, 0)),
            pl.BlockSpec((1, F), lambda b, mb: (0, 0)),
            pl.BlockSpec((F, 1), lambda b, mb: (0, 0)),
            pl.BlockSpec((1, 1), lambda b, mb: (0, 0)),
            pl.BlockSpec((1, 1, L), lambda b, mb: (b, 0, 0)),
            pl.BlockSpec(memory_space=pltpu.SMEM),
        ],
        out_specs=[
            pl.BlockSpec((1, L, 1), lambda b, mb: (b, 0, 0)),
            pl.BlockSpec((1, MBLK, D), lambda b, mb: (b, mb, 0)),
        ],
        out_shape=[
            jax.ShapeDtypeStruct((B, L, 1), jnp.float32),
            jax.ShapeDtypeStruct((B, M, D), jnp.float32),
        ],
        scratch_shapes=[pltpu.VMEM((1, L), jnp.float32)],
    )(x, conv1_w, row(conv1_b), row(ln1_g), row(ln1_b), conv2_w,
      row(conv2_b), row(ln2_g), row(ln2_b), lin_w, lin_b.reshape(1, 1),
      target.reshape(B, 1, L),
      jnp.sum(target, axis=1, dtype=jnp.int32))
    return (out, dur3.reshape(B, L))


# speedup vs baseline: 1.6031x; 1.0517x over previous
import jax
import jax.numpy as jnp
from jax import lax
from jax.experimental import pallas as pl
from jax.experimental.pallas import tpu as pltpu

B, L, D = 8, 512, 256
F, K, M = 256, 3, 4096
NMB = 1
MBLK = M // NMB
SUB = 2048


def _ln_relu(h, g, bb):
    mu = jnp.mean(h, axis=-1, keepdims=True)
    var = jnp.mean((h - mu) ** 2, axis=-1, keepdims=True)
    hn = (h - mu) * lax.rsqrt(var + 1e-5) * g + bb
    return jnp.maximum(hn, 0.0)


def _conv3(h, w_ref, bias):
    hb = h.astype(jnp.bfloat16)
    y0 = jnp.dot(hb, w_ref[0].astype(jnp.bfloat16),
                 preferred_element_type=jnp.float32)
    y1 = jnp.dot(hb, w_ref[1].astype(jnp.bfloat16),
                 preferred_element_type=jnp.float32)
    y2 = jnp.dot(hb, w_ref[2].astype(jnp.bfloat16),
                 preferred_element_type=jnp.float32)
    z = jnp.zeros((1, y0.shape[1]), jnp.float32)
    return (jnp.concatenate([z, y0[:-1]], axis=0)
            + y1
            + jnp.concatenate([y2[1:], z], axis=0)
            + bias)


def _tc_body(x_ref, w1_ref, b1_ref, g1_ref, bb1_ref, w2_ref, b2_ref,
             g2_ref, bb2_ref, lw_ref, lb_ref, tgt_ref, tot_ref,
             dur_ref, out_ref, csum_ref):
    b = pl.program_id(0)
    mb = pl.program_id(1)
    xb = x_ref[0]

    @pl.when(mb == 0)
    def _predictor():
        h = _conv3(xb, w1_ref, b1_ref[...])
        h = _ln_relu(h, g1_ref[...], bb1_ref[...])
        h = _conv3(h, w2_ref, b2_ref[...])
        h = _ln_relu(h, g2_ref[...], bb2_ref[...])
        dur = jnp.maximum(
            jnp.dot(h, lw_ref[...], preferred_element_type=jnp.float32)
            + lb_ref[...], 0.0)
        dur_ref[0] = dur
        tgt = tgt_ref[0].astype(jnp.float32)
        ik = lax.broadcasted_iota(jnp.int32, (L, L), 0)
        ij = lax.broadcasted_iota(jnp.int32, (L, L), 1)
        lower_tri = (ik <= ij).astype(jnp.float32)
        csum_ref[...] = jnp.dot(tgt, lower_tri,
                                preferred_element_type=jnp.float32)

    tot = tot_ref[b]
    xb16 = xb.astype(jnp.bfloat16)
    csum = csum_ref[...]
    for s in range(MBLK // SUB):
        t0 = mb * MBLK + s * SUB

        @pl.when(t0 < tot)
        def _expand():
            t_col = (lax.broadcasted_iota(jnp.int32, (SUB, 1), 0)
                     + t0).astype(jnp.float32)
            v = (t_col < csum).astype(jnp.bfloat16)
            z = jnp.zeros((SUB, 1), jnp.bfloat16)
            onehot = v - jnp.concatenate([z, v[:, :-1]], axis=1)
            out_ref[0, pl.ds(s * SUB, SUB), :] = jnp.dot(
                onehot, xb16, preferred_element_type=jnp.float32)

        @pl.when(t0 >= tot)
        def _zero():
            out_ref[0, pl.ds(s * SUB, SUB), :] = jnp.zeros(
                (SUB, D), jnp.float32)


def kernel(x, conv1_w, conv1_b, ln1_g, ln1_b, conv2_w, conv2_b, ln2_g, ln2_b,
           lin_w, lin_b, target, mel_max_length):
    del mel_max_length
    row = lambda v: v.reshape(1, -1)
    dur3, out = pl.pallas_call(
        _tc_body,
        grid=(B, NMB),
        in_specs=[
            pl.BlockSpec((1, L, D), lambda b, mb: (b, 0, 0)),
            pl.BlockSpec((K, D, F), lambda b, mb: (0, 0, 0)),
            pl.BlockSpec((1, F), lambda b, mb: (0, 0)),
            pl.BlockSpec((1, F), lambda b, mb: (0, 0)),
            pl.BlockSpec((1, F), lambda b, mb: (0, 0)),
            pl.BlockSpec((K, F, F), lambda b, mb: (0, 0, 0)),
            pl.BlockSpec((1, F), lambda b, mb: (0, 0)),
            pl.BlockSpec((1, F), lambda b, mb: (0, 0)),
            pl.BlockSpec((1, F), lambda b, mb: (0, 0)),
            pl.BlockSpec((F, 1), lambda b, mb: (0, 0)),
            pl.BlockSpec((1, 1), lambda b, mb: (0, 0)),
            pl.BlockSpec((1, 1, L), lambda b, mb: (b, 0, 0)),
            pl.BlockSpec(memory_space=pltpu.SMEM),
        ],
        out_specs=[
            pl.BlockSpec((1, L, 1), lambda b, mb: (b, 0, 0)),
            pl.BlockSpec((1, MBLK, D), lambda b, mb: (b, mb, 0)),
        ],
        out_shape=[
            jax.ShapeDtypeStruct((B, L, 1), jnp.float32),
            jax.ShapeDtypeStruct((B, M, D), jnp.float32),
        ],
        scratch_shapes=[pltpu.VMEM((1, L), jnp.float32)],
    )(x, conv1_w, row(conv1_b), row(ln1_g), row(ln1_b), conv2_w,
      row(conv2_b), row(ln2_g), row(ln2_b), lin_w, lin_b.reshape(1, 1),
      target.reshape(B, 1, L),
      jnp.sum(target, axis=1, dtype=jnp.int32))
    return (out, dur3.reshape(B, L))
